# Initial kernel scaffold; baseline (speedup 1.0000x reference)
import jax
import jax.numpy as jnp
from jax.experimental import pallas as pl


def kernel(points, cells, src, dst, bcells, bpoints):
    raise NotImplementedError("write your pallas kernel here")


# speedup vs baseline: 41.0797x; 41.0797x over previous
import jax
import jax.numpy as jnp
from jax import lax
from jax.experimental import pallas as pl
from jax.experimental.pallas import tpu as pltpu
from jax.experimental.pallas import tpu_sc as plsc

NC = 2
NS = 16
NW = NC * NS
L = 16

KC = 80
KE = 128
KB = 80


def _iota16():
    return lax.broadcasted_iota(jnp.int32, (L,), 0)


def _face_norm(a1, a2, b1, b2, c1, c2):
    s = jnp.sign((b1 - c1) * (a2 - c2) - (b2 - c2) * (a1 - c1))
    return s * (b2 - c2), -s * (b1 - c1)


def _alpha(x1, y1, x2, y2, x3, y3, x4, y4):
    y21 = y2 - y1
    y43 = y4 - y3
    y31 = y3 - y1
    x31 = x3 - x1
    x21 = x2 - x1
    x43 = x4 - x3
    return (x31 * y43 - y31 * x43) / (x21 * y43 - y21 * x43)


def _col(ref, row, k):
    return plsc.load_gather(ref, [row, jnp.full((L,), k, jnp.int32)])


def _cell_body(c0, c1, c2, px, py,
               packed, xo, yo, ao, ho,
               c0v, c1v, c2v,
               g0x, g0y, g1x, g1y, g2x, g2y,
               pk, xv, yv, av, hv, sem):
    wid = lax.axis_index("s") * NC + lax.axis_index("c")
    nchunks = c0.shape[0] // KC
    q, r = nchunks // NW, nchunks % NW
    count = q + (wid < r).astype(jnp.int32)
    iota = _iota16()

    def body(j, carry):
        c = wid + j * NW
        base = pl.multiple_of(c * KC, 8)
        pltpu.sync_copy(c0.at[pl.ds(base, KC)], c0v)
        pltpu.sync_copy(c1.at[pl.ds(base, KC)], c1v)
        pltpu.sync_copy(c2.at[pl.ds(base, KC)], c2v)
        cps = [
            pltpu.async_copy(px.at[c0v], g0x, sem),
            pltpu.async_copy(py.at[c0v], g0y, sem),
            pltpu.async_copy(px.at[c1v], g1x, sem),
            pltpu.async_copy(py.at[c1v], g1y, sem),
            pltpu.async_copy(px.at[c2v], g2x, sem),
            pltpu.async_copy(py.at[c2v], g2y, sem),
        ]
        for cp in cps:
            cp.wait()
        for g in range(KC // L):
            sl = pl.ds(g * L, L)
            x0 = g0x[sl]
            y0 = g0y[sl]
            x1 = g1x[sl]
            y1 = g1y[sl]
            x2 = g2x[sl]
            y2 = g2y[sl]
            cx = (x0 + x1 + x2) / 3.0
            cy = (y0 + y1 + y2) / 3.0
            area = 0.5 * jnp.abs(x0 * (y1 - y2) + x1 * (y2 - y0)
                                 + x2 * (y0 - y1))
            hh = 1.0 + 0.1 * jnp.exp(-100.0 * (cx * cx + cy * cy))
            rowbase = (g * L + iota) * 8
            for k, val in enumerate((x0, y0, x1, y1, x2, y2, cx, cy)):
                plsc.store_scatter(pk, [rowbase + k], val)
            xv[sl] = cx
            yv[sl] = cy
            av[sl] = area
            hv[sl] = hh
        pltpu.sync_copy(pk, packed.at[pl.ds(base * 8, KC * 8)])
        pltpu.sync_copy(xv, xo.at[pl.ds(base, KC)])
        pltpu.sync_copy(yv, yo.at[pl.ds(base, KC)])
        pltpu.sync_copy(av, ao.at[pl.ds(base, KC)])
        pltpu.sync_copy(hv, ho.at[pl.ds(base, KC)])
        return carry

    lax.fori_loop(0, count, body, 0)


def _edge_body(src, dst, packed, px, py, b0, b1, bc,
               sxo, syo, alo, bsxo, bsyo,
               sidx, didx, srow, drow, sxv, syv, alv,
               b0v, b1v, bcv, p0xv, p0yv, p1xv, p1yv, brow, bsxv, bsyv,
               sem):
    wid = lax.axis_index("s") * NC + lax.axis_index("c")
    iota = _iota16()
    zero = jnp.zeros((L,), jnp.float32)

    nchunks = src.shape[0] // KE
    q, r = nchunks // NW, nchunks % NW
    count = q + (wid < r).astype(jnp.int32)

    def body(j, carry):
        c = wid + j * NW
        base = pl.multiple_of(c * KE, 8)
        pltpu.sync_copy(src.at[pl.ds(base, KE)], sidx)
        pltpu.sync_copy(dst.at[pl.ds(base, KE)], didx)
        cps = [
            pltpu.async_copy(packed.at[sidx], srow, sem),
            pltpu.async_copy(packed.at[didx], drow, sem),
        ]
        for cp in cps:
            cp.wait()
        for g in range(KE // L):
            sl = pl.ds(g * L, L)
            row = g * L + iota
            a0x = _col(srow, row, 0)
            a0y = _col(srow, row, 1)
            a1x = _col(srow, row, 2)
            a1y = _col(srow, row, 3)
            a2x = _col(srow, row, 4)
            a2y = _col(srow, row, 5)
            scx = _col(srow, row, 6)
            scy = _col(srow, row, 7)
            dcx = _col(drow, row, 6)
            dcy = _col(drow, row, 7)

            n01x, n01y = _face_norm(a2x, a2y, a0x, a0y, a1x, a1y)
            n12x, n12y = _face_norm(a0x, a0y, a1x, a1y, a2x, a2y)
            n20x, n20y = _face_norm(a1x, a1y, a2x, a2y, a0x, a0y)
            m01x, m01y = _face_norm(dcx, dcy, a0x, a0y, a1x, a1y)
            m12x, m12y = _face_norm(dcx, dcy, a1x, a1y, a2x, a2y)
            m20x, m20y = _face_norm(dcx, dcy, a2x, a2y, a0x, a0y)

            cond = n01x * m01x + n01y * m01y < 0
            sx = jnp.where(cond, n01x, zero)
            sy = jnp.where(cond, n01y, zero)
            al = _alpha(scx, scy, dcx, dcy, a0x, a0y, a1x, a1y)
            alp = jnp.where(cond, al, zero)
            cond = n12x * m12x + n12y * m12y < 0
            sx = jnp.where(cond, n12x, sx)
            sy = jnp.where(cond, n12y, sy)
            al = _alpha(scx, scy, dcx, dcy, a1x, a1y, a2x, a2y)
            alp = jnp.where(cond, al, alp)
            cond = n20x * m20x + n20y * m20y < 0
            sx = jnp.where(cond, n20x, sx)
            sy = jnp.where(cond, n20y, sy)
            al = _alpha(scx, scy, dcx, dcy, a2x, a2y, a0x, a0y)
            alp = jnp.where(cond, al, alp)

            sxv[sl] = sx
            syv[sl] = sy
            alv[sl] = alp
        pltpu.sync_copy(sxv, sxo.at[pl.ds(base, KE)])
        pltpu.sync_copy(syv, syo.at[pl.ds(base, KE)])
        pltpu.sync_copy(alv, alo.at[pl.ds(base, KE)])
        return carry

    lax.fori_loop(0, count, body, 0)

    nbch = bc.shape[0] // KB
    qb, rb = nbch // NW, nbch % NW
    countb = qb + (wid < rb).astype(jnp.int32)

    def bbody(j, carry):
        c = wid + j * NW
        base = pl.multiple_of(c * KB, 8)
        pltpu.sync_copy(b0.at[pl.ds(base, KB)], b0v)
        pltpu.sync_copy(b1.at[pl.ds(base, KB)], b1v)
        pltpu.sync_copy(bc.at[pl.ds(base, KB)], bcv)
        cps = [
            pltpu.async_copy(px.at[b0v], p0xv, sem),
            pltpu.async_copy(py.at[b0v], p0yv, sem),
            pltpu.async_copy(px.at[b1v], p1xv, sem),
            pltpu.async_copy(py.at[b1v], p1yv, sem),
            pltpu.async_copy(packed.at[bcv], brow, sem),
        ]
        for cp in cps:
            cp.wait()
        for g in range(KB // L):
            sl = pl.ds(g * L, L)
            row = g * L + iota
            ccx = _col(brow, row, 6)
            ccy = _col(brow, row, 7)
            bnx, bny = _face_norm(ccx, ccy, p0xv[sl], p0yv[sl],
                                  p1xv[sl], p1yv[sl])
            bsxv[sl] = -bnx
            bsyv[sl] = -bny
        pltpu.sync_copy(bsxv, bsxo.at[pl.ds(base, KB)])
        pltpu.sync_copy(bsyv, bsyo.at[pl.ds(base, KB)])
        return carry

    lax.fori_loop(0, countb, bbody, 0)


def kernel(points, cells, src, dst, bcells, bpoints):
    C = cells.shape[0]
    E = src.shape[0]
    B = bcells.shape[0]
    f32 = jnp.float32

    px = points[:, 0]
    py = points[:, 1]
    c0 = cells[:, 0]
    c1 = cells[:, 1]
    c2 = cells[:, 2]
    b0 = bpoints[:, 0]
    b1 = bpoints[:, 1]

    mesh = plsc.VectorSubcoreMesh(core_axis_name="c", subcore_axis_name="s")

    cell_k = pl.kernel(
        _cell_body,
        out_type=(
            jax.ShapeDtypeStruct((C * 8,), f32),
            jax.ShapeDtypeStruct((C,), f32),
            jax.ShapeDtypeStruct((C,), f32),
            jax.ShapeDtypeStruct((C,), f32),
            jax.ShapeDtypeStruct((C,), f32),
        ),
        mesh=mesh,
        compiler_params=pltpu.CompilerParams(needs_layout_passes=False, use_tc_tiling_on_sc=False),
        scratch_types=[
            pltpu.VMEM((KC,), jnp.int32),
            pltpu.VMEM((KC,), jnp.int32),
            pltpu.VMEM((KC,), jnp.int32),
            pltpu.VMEM((KC,), f32),
            pltpu.VMEM((KC,), f32),
            pltpu.VMEM((KC,), f32),
            pltpu.VMEM((KC,), f32),
            pltpu.VMEM((KC,), f32),
            pltpu.VMEM((KC,), f32),
            pltpu.VMEM((KC * 8,), f32),
            pltpu.VMEM((KC,), f32),
            pltpu.VMEM((KC,), f32),
            pltpu.VMEM((KC,), f32),
            pltpu.VMEM((KC,), f32),
            pltpu.SemaphoreType.DMA,
        ],
    )
    packed, x, y, area, h = cell_k(c0, c1, c2, px, py)
    packed = packed.reshape(C, 8)

    edge_k = pl.kernel(
        _edge_body,
        out_type=(
            jax.ShapeDtypeStruct((E,), f32),
            jax.ShapeDtypeStruct((E,), f32),
            jax.ShapeDtypeStruct((E,), f32),
            jax.ShapeDtypeStruct((B,), f32),
            jax.ShapeDtypeStruct((B,), f32),
        ),
        mesh=mesh,
        compiler_params=pltpu.CompilerParams(needs_layout_passes=False, use_tc_tiling_on_sc=False),
        scratch_types=[
            pltpu.VMEM((KE,), jnp.int32),
            pltpu.VMEM((KE,), jnp.int32),
            pltpu.VMEM((KE, 8), f32),
            pltpu.VMEM((KE, 8), f32),
            pltpu.VMEM((KE,), f32),
            pltpu.VMEM((KE,), f32),
            pltpu.VMEM((KE,), f32),
            pltpu.VMEM((KB,), jnp.int32),
            pltpu.VMEM((KB,), jnp.int32),
            pltpu.VMEM((KB,), jnp.int32),
            pltpu.VMEM((KB,), f32),
            pltpu.VMEM((KB,), f32),
            pltpu.VMEM((KB,), f32),
            pltpu.VMEM((KB,), f32),
            pltpu.VMEM((KB, 8), f32),
            pltpu.VMEM((KB,), f32),
            pltpu.VMEM((KB,), f32),
            pltpu.SemaphoreType.DMA,
        ],
    )
    sx, sy, alpha, bsx, bsy = edge_k(src, dst, packed, px, py, b0, b1, bcells)

    return (x, y, area, sx, sy, bsx, bsy, h, alpha)
